# baseline (speedup 1.0000x reference)
import functools

import jax
import jax.numpy as jnp
from jax import lax
from jax.experimental import pallas as pl
from jax.experimental.pallas import tpu as pltpu
from jax.experimental.pallas import tpu_sc as plsc

B = 16384
D = 16
L = 16
NC = 2
NS = 16
NW = NC * NS
BPW = B // NW
CHUNK = 128
NCH = BPW // CHUNK
NGRP = BPW // L

def _bpr_sc_body(
    uid_hbm,
    pid_hbm,
    nid_hbm,
    ut_hbm,
    pt_hbm,
    pos_out,
    neg_out,
    uid_v,
    pid_v,
    nid_v,
    u_rows,
    i_rows,
    j_rows,
    ps_v,
    ns_v,
    sem,
):
    wid = lax.axis_index("s") * NC + lax.axis_index("c")
    rbase = wid * NCH
    base = wid * BPW

    pltpu.sync_copy(uid_hbm.at[pl.ds(rbase, NCH)], uid_v)
    pltpu.sync_copy(pid_hbm.at[pl.ds(rbase, NCH)], pid_v)
    pltpu.sync_copy(nid_hbm.at[pl.ds(rbase, NCH)], nid_v)

    copies = []
    for k in range(NCH):
        dst = pl.ds(k * CHUNK, CHUNK)
        copies.append(pltpu.async_copy(ut_hbm.at[uid_v.at[k]], u_rows.at[dst], sem))
        copies.append(pltpu.async_copy(pt_hbm.at[pid_v.at[k]], i_rows.at[dst], sem))
        copies.append(pltpu.async_copy(pt_hbm.at[nid_v.at[k]], j_rows.at[dst], sem))
    for cp in copies:
        cp.wait()

    lane = lax.iota(jnp.int32, L)

    def group(g, carry):
        rows = g * L + lane
        accp = jnp.zeros((L,), jnp.float32)
        accn = jnp.zeros((L,), jnp.float32)
        for c in range(D):
            cc = jnp.full((L,), c, jnp.int32)
            uc = plsc.load_gather(u_rows, [rows, cc])
            ic = plsc.load_gather(i_rows, [rows, cc])
            jc = plsc.load_gather(j_rows, [rows, cc])
            accp = accp + uc * ic
            accn = accn + uc * jc
        ps_v[pl.ds(g * L, L)] = accp
        ns_v[pl.ds(g * L, L)] = accn
        return carry

    lax.fori_loop(0, NGRP, group, 0)

    pltpu.sync_copy(ps_v, pos_out.at[pl.ds(base, BPW)])
    pltpu.sync_copy(ns_v, neg_out.at[pl.ds(base, BPW)])


@functools.cache
def _build():
    mesh = plsc.VectorSubcoreMesh(
        core_axis_name="c", subcore_axis_name="s", num_cores=NC, num_subcores=NS
    )
    return pl.kernel(
        _bpr_sc_body,
        out_type=(
            jax.ShapeDtypeStruct((B,), jnp.float32),
            jax.ShapeDtypeStruct((B,), jnp.float32),
        ),
        mesh=mesh,
        scratch_types=[
            pltpu.VMEM((NCH, CHUNK), jnp.int32),
            pltpu.VMEM((NCH, CHUNK), jnp.int32),
            pltpu.VMEM((NCH, CHUNK), jnp.int32),
            pltpu.VMEM((BPW, D), jnp.float32),
            pltpu.VMEM((BPW, D), jnp.float32),
            pltpu.VMEM((BPW, D), jnp.float32),
            pltpu.VMEM((BPW,), jnp.float32),
            pltpu.VMEM((BPW,), jnp.float32),
            pltpu.SemaphoreType.DMA,
        ],
        compiler_params=pltpu.CompilerParams(
            needs_layout_passes=False, use_tc_tiling_on_sc=False
        ),
    )


def kernel(user_ids, pos_pids, neg_pids, user_table, playlist_table):
    uid2 = user_ids.astype(jnp.int32).reshape(B // CHUNK, CHUNK)
    pid2 = pos_pids.astype(jnp.int32).reshape(B // CHUNK, CHUNK)
    nid2 = neg_pids.astype(jnp.int32).reshape(B // CHUNK, CHUNK)
    pos, neg = _build()(uid2, pid2, nid2, user_table, playlist_table)
    return (pos, neg)
